# baseline (speedup 1.0000x reference)
import functools

import jax
import jax.numpy as jnp
from jax import lax
from jax.experimental import pallas as pl
from jax.experimental.pallas import tpu as pltpu
from jax.experimental.pallas import tpu_sc as plsc

LANES = 16


def _sc_add_kernel(B, N, T, D, NC=1, NS=16):
    NW = NC * NS
    BN = B * N
    assert BN % NW == 0
    V_PER_W = BN // NW
    assert N % V_PER_W == 0
    WPB = N // V_PER_W
    NV = 25
    assert V_PER_W % NV == 0
    NCHUNK = V_PER_W // NV
    assert NCHUNK % 2 == 0
    assert D % LANES == 0
    G = D // LANES

    mesh = plsc.VectorSubcoreMesh(core_axis_name="c", subcore_axis_name="s", num_cores=1)

    @functools.partial(
        pl.kernel,
        out_type=jax.ShapeDtypeStruct((B, N, T, D), jnp.float32),
        mesh=mesh,
        compiler_params=pltpu.CompilerParams(use_tc_tiling_on_sc=True),
        scratch_types=[
            pltpu.VMEM((2, NV, T, D), jnp.float32),
            pltpu.VMEM((2, NV * D), jnp.float32),
            pltpu.SemaphoreType.DMA,
            pltpu.SemaphoreType.DMA,
            pltpu.SemaphoreType.DMA,
            pltpu.SemaphoreType.DMA,
        ],
    )
    def body(x_hbm, w_hbm, out_hbm, xbuf, wbuf, si0, si1, so0, so1):
        wid = lax.axis_index("s") * NC + lax.axis_index("c")
        b = wid // WPB
        n_base = (wid % WPB) * V_PER_W
        sins = (si0, si1)
        souts = (so0, so1)

        def in_descs(i, r):
            n0 = n_base + i * NV
            dx = pltpu.make_async_copy(
                x_hbm.at[b, pl.ds(n0, NV)], xbuf.at[r], sins[r])
            dw = pltpu.make_async_copy(
                w_hbm.at[pl.ds(n0 * D, NV * D)], wbuf.at[r], sins[r])
            return dx, dw

        def start_in(i, r):
            dx, dw = in_descs(i, r)
            dx.start()
            dw.start()

        def wait_in(i, r):
            dx, dw = in_descs(i, r)
            dx.wait()
            dw.wait()

        def out_desc(i, r):
            n0 = n_base + i * NV
            return pltpu.make_async_copy(
                xbuf.at[r], out_hbm.at[b, pl.ds(n0, NV)], souts[r])

        def compute(r):
            def vert(v, _):
                for g in range(G):
                    wv = wbuf[r, pl.ds(v * D + g * LANES, LANES)]
                    for t in range(T):
                        sl = (r, v, t, pl.ds(g * LANES, LANES))
                        xbuf[sl] = xbuf[sl] + wv
                return 0

            lax.fori_loop(0, NV, vert, 0)

        def process(i, r):
            wait_in(i, r)
            compute(r)
            out_desc(i, r).start()

        start_in(0, 0)
        start_in(1, 1)

        def pair(k, _):
            i0 = 2 * k
            process(i0, 0)
            process(i0 + 1, 1)
            out_desc(i0, 0).wait()
            start_in(i0 + 2, 0)
            out_desc(i0 + 1, 1).wait()
            start_in(i0 + 3, 1)
            return 0

        lax.fori_loop(0, NCHUNK // 2 - 1, pair, 0)
        i0 = NCHUNK - 2
        process(i0, 0)
        process(i0 + 1, 1)
        out_desc(i0, 0).wait()
        out_desc(i0 + 1, 1).wait()

    return body


def kernel(x, embedding_weight):
    B, N, T, D = x.shape
    fn = _sc_add_kernel(B, N, T, D)
    return fn(x, embedding_weight.reshape(-1))


# speedup vs baseline: 1.3330x; 1.3330x over previous
import functools

import jax
import jax.numpy as jnp
from jax import lax
from jax.experimental import pallas as pl
from jax.experimental.pallas import tpu as pltpu
from jax.experimental.pallas import tpu_sc as plsc

LANES = 16


def _sc_add_kernel(B, N, T, D, NC=2, NS=16):
    NW = NC * NS
    BN = B * N
    assert BN % NW == 0
    V_PER_W = BN // NW
    assert N % V_PER_W == 0
    WPB = N // V_PER_W
    NV = 25
    assert V_PER_W % NV == 0
    NCHUNK = V_PER_W // NV
    assert NCHUNK % 2 == 0
    assert D % LANES == 0
    G = D // LANES

    mesh = plsc.VectorSubcoreMesh(core_axis_name="c", subcore_axis_name="s")

    @functools.partial(
        pl.kernel,
        out_type=jax.ShapeDtypeStruct((B, N, T, D), jnp.float32),
        mesh=mesh,
        compiler_params=pltpu.CompilerParams(use_tc_tiling_on_sc=True),
        scratch_types=[
            pltpu.VMEM((2, NV, T, D), jnp.float32),
            pltpu.VMEM((2, NV * D), jnp.float32),
            pltpu.SemaphoreType.DMA,
            pltpu.SemaphoreType.DMA,
            pltpu.SemaphoreType.DMA,
            pltpu.SemaphoreType.DMA,
        ],
    )
    def body(x_hbm, w_hbm, out_hbm, xbuf, wbuf, si0, si1, so0, so1):
        wid = lax.axis_index("s") * NC + lax.axis_index("c")
        b = wid // WPB
        n_base = (wid % WPB) * V_PER_W
        sins = (si0, si1)
        souts = (so0, so1)

        def in_descs(i, r):
            n0 = n_base + i * NV
            dx = pltpu.make_async_copy(
                x_hbm.at[b, pl.ds(n0, NV)], xbuf.at[r], sins[r])
            dw = pltpu.make_async_copy(
                w_hbm.at[pl.ds(n0 * D, NV * D)], wbuf.at[r], sins[r])
            return dx, dw

        def start_in(i, r):
            dx, dw = in_descs(i, r)
            dx.start()
            dw.start()

        def wait_in(i, r):
            dx, dw = in_descs(i, r)
            dx.wait()
            dw.wait()

        def out_desc(i, r):
            n0 = n_base + i * NV
            return pltpu.make_async_copy(
                xbuf.at[r], out_hbm.at[b, pl.ds(n0, NV)], souts[r])

        def compute(r):
            @plsc.parallel_loop(0, NV, 1, unroll=2)
            def vert(v):
                for g in range(G):
                    wv = wbuf[r, pl.ds(v * D + g * LANES, LANES)]
                    for t in range(T):
                        sl = (r, v, t, pl.ds(g * LANES, LANES))
                        xbuf[sl] = xbuf[sl] + wv

        def process(i, r):
            wait_in(i, r)
            compute(r)
            out_desc(i, r).start()

        start_in(0, 0)
        start_in(1, 1)

        def pair(k, _):
            i0 = 2 * k
            process(i0, 0)
            process(i0 + 1, 1)
            out_desc(i0, 0).wait()
            start_in(i0 + 2, 0)
            out_desc(i0 + 1, 1).wait()
            start_in(i0 + 3, 1)
            return 0

        lax.fori_loop(0, NCHUNK // 2 - 1, pair, 0)
        i0 = NCHUNK - 2
        process(i0, 0)
        process(i0 + 1, 1)
        out_desc(i0, 0).wait()
        out_desc(i0 + 1, 1).wait()

    return body


def kernel(x, embedding_weight):
    B, N, T, D = x.shape
    fn = _sc_add_kernel(B, N, T, D)
    return fn(x, embedding_weight.reshape(-1))
